# baseline (speedup 1.0000x reference)
import functools

import jax
import jax.numpy as jnp
from jax import lax
from jax.experimental import pallas as pl
from jax.experimental.pallas import tpu as pltpu
from jax.experimental.pallas import tpu_sc as plsc

_E = 8
_H = 1024
_F = 1024
_TOPK = 2
_S = 2048
_A = _S * _TOPK

_BLK = 128
_NPAD = 5120
_NBLK = _NPAD // _BLK

_NTILES = 32
_RPT = _NPAD // _NTILES
_GCH = 16
_NGCH = _RPT // _GCH
_GNB = 4
_TPT = _S // _NTILES
_CCH = 16
_NCCH = _TPT // _CCH
_LANES = 16


def _route(top_k_index, top_k_weights):
    flat_e = top_k_index.reshape(-1).astype(jnp.int32)
    order = jnp.argsort(flat_e).astype(jnp.int32)
    rank = jnp.argsort(order).astype(jnp.int32)
    w_sorted = top_k_weights.reshape(-1).astype(jnp.float32)[order]

    eids = jnp.arange(_E, dtype=jnp.int32)
    cnt = jnp.sum(flat_e[None, :] == eids[:, None], axis=1).astype(jnp.int32)
    start = jnp.concatenate([jnp.zeros(1, jnp.int32), jnp.cumsum(cnt)[:-1]])
    bcnt = (cnt + _BLK - 1) // _BLK
    bends = jnp.cumsum(bcnt)
    bstart = jnp.concatenate([jnp.zeros(1, jnp.int32), bends[:-1]])
    pstart = bstart * _BLK
    pends = bends * _BLK

    pos_orig = jnp.take(pstart - start, flat_e) + rank

    used = bends[-1]
    bids = jnp.arange(_NBLK, dtype=jnp.int32)
    be_raw = jnp.minimum(jnp.sum(bids[None, :] >= bends[:, None], axis=0),
                         _E - 1).astype(jnp.int32)

    r2d = jnp.arange(_NPAD, dtype=jnp.int32).reshape(_NBLK, _BLK)
    a2d = r2d - jnp.take(pstart - start, be_raw)[:, None]
    valid2d = a2d < jnp.take(start + cnt, be_raw)[:, None]
    a2d = jnp.clip(a2d, 0, _A - 1)
    a_enc = jnp.where(valid2d, a2d,
                      -jnp.bitwise_and(r2d, _S - 1) - 1).reshape(_NPAD)
    be_last = jnp.take(be_raw, used - 1)
    be = jnp.where(bids < used, be_raw, be_last)
    xi = jnp.where(bids < used, bids, used - 1).astype(jnp.int32)
    vld = (bids < used).astype(jnp.int32)
    meta = jnp.stack([be, xi, vld])
    return order, w_sorted, a_enc, pos_orig, meta


@functools.cache
def _gather_rows_kernel():
    mesh = plsc.VectorSubcoreMesh(core_axis_name="c", subcore_axis_name="s")

    @functools.partial(
        pl.kernel,
        mesh=mesh,
        out_type=(
            jax.ShapeDtypeStruct((_NPAD, _F), jnp.float32),
            jax.ShapeDtypeStruct((_NPAD,), jnp.float32),
        ),
        scratch_types=(
            [
                pltpu.VMEM((_RPT,), jnp.int32),
                pltpu.VMEM((_RPT,), jnp.int32),
                pltpu.VMEM((_RPT,), jnp.int32),
                pltpu.VMEM((_RPT,), jnp.float32),
                pltpu.VMEM((_NGCH, _GCH), jnp.int32),
                pltpu.VMEM((_RPT,), jnp.float32),
            ]
            + [pltpu.VMEM((_GCH, _F), jnp.float32) for _ in range(_GNB)]
            + [pltpu.SemaphoreType.DMA for _ in range(2 + 2 * _GNB)]
        ),
    )
    def _gather_rows(aenc_hbm, order_hbm, ws_hbm, hs_hbm, x_hbm, rw_hbm,
                     aenc_v, apos_v, o_v, wv_v, idx_v, rw_v, *bufsem):
        bufs = bufsem[:_GNB]
        se0, se1 = bufsem[_GNB], bufsem[_GNB + 1]
        sgs = bufsem[_GNB + 2:2 * _GNB + 2]
        sws = bufsem[2 * _GNB + 2:]
        wid = lax.axis_index("s") * 2 + lax.axis_index("c")
        base = wid * _RPT

        pltpu.sync_copy(aenc_hbm.at[pl.ds(base, _RPT)], aenc_v)
        for c in range(_NGCH):
            sl = pl.ds(c * _GCH, _GCH)
            apos_v[sl] = jnp.maximum(aenc_v[sl], 0)
        cp0 = pltpu.async_copy(order_hbm.at[apos_v], o_v, se0)
        cp1 = pltpu.async_copy(ws_hbm.at[apos_v], wv_v, se1)
        cp0.wait()
        cp1.wait()
        for c in range(_NGCH):
            sl = pl.ds(c * _GCH, _GCH)
            av = aenc_v[sl]
            vld = av >= 0
            idx_v[c, :] = jnp.where(vld, jnp.right_shift(o_v[sl], 1), -av - 1)
            rw_v[sl] = jnp.where(vld, wv_v[sl], 0.0)
        pltpu.sync_copy(rw_v, rw_hbm.at[pl.ds(base, _RPT)])

        gat = [None] * _GNB
        wrt = [None] * _GNB

        def _write(c):
            q = c % _GNB
            gat[q].wait()
            wrt[q] = pltpu.async_copy(
                bufs[q], x_hbm.at[pl.ds(base + c * _GCH, _GCH)], sws[q])

        for c in range(_NGCH):
            p = c % _GNB
            if wrt[p] is not None:
                wrt[p].wait()
            gat[p] = pltpu.async_copy(hs_hbm.at[idx_v.at[c]], bufs[p], sgs[p])
            if c >= 2:
                _write(c - 2)
        _write(_NGCH - 2)
        _write(_NGCH - 1)
        for p in range(_GNB):
            if wrt[p] is not None:
                wrt[p].wait()

    return _gather_rows


def _expert_body(meta_ref, x_ref, w1_ref, v1_ref, w2_ref, rw_ref, y_ref):
    i = pl.program_id(0)

    @pl.when(meta_ref[2, i] == 1)
    def _():
        x = x_ref[...]
        g = jnp.dot(x, w1_ref[0], preferred_element_type=jnp.float32)
        u = jnp.dot(x, v1_ref[0], preferred_element_type=jnp.float32)
        inter = (g * jax.nn.sigmoid(g)) * u
        y = lax.dot_general(inter, w2_ref[0], (((1,), (1,)), ((), ())),
                            preferred_element_type=jnp.float32)
        y_ref[...] = y * rw_ref[...]


def _expert_matmul(meta, x, w1r, v1r, w2r, rw):
    grid_spec = pltpu.PrefetchScalarGridSpec(
        num_scalar_prefetch=1,
        grid=(_NBLK,),
        in_specs=[
            pl.BlockSpec((_BLK, _F), lambda i, m: (m[1, i], 0)),
            pl.BlockSpec((1, _F, _H), lambda i, m: (m[0, i], 0, 0)),
            pl.BlockSpec((1, _F, _H), lambda i, m: (m[0, i], 0, 0)),
            pl.BlockSpec((1, _F, _H), lambda i, m: (m[0, i], 0, 0)),
            pl.BlockSpec((_BLK, 1), lambda i, m: (m[1, i], 0)),
        ],
        out_specs=pl.BlockSpec((_BLK, _F), lambda i, m: (i, 0)),
    )
    return pl.pallas_call(
        _expert_body,
        grid_spec=grid_spec,
        out_shape=jax.ShapeDtypeStruct((_NPAD, _F), jnp.float32),
    )(meta, x, w1r, v1r, w2r, rw)


@functools.cache
def _combine_kernel():
    mesh = plsc.VectorSubcoreMesh(core_axis_name="c", subcore_axis_name="s")

    @functools.partial(
        pl.kernel,
        mesh=mesh,
        out_type=jax.ShapeDtypeStruct((_S, _F), jnp.float32),
        scratch_types=[
            pltpu.VMEM((_NCCH, _CCH), jnp.int32),
            pltpu.VMEM((_NCCH, _CCH), jnp.int32),
            pltpu.VMEM((_CCH, _F), jnp.float32),
            pltpu.VMEM((_CCH, _F), jnp.float32),
            pltpu.VMEM((_CCH, _F), jnp.float32),
            pltpu.VMEM((_CCH, _F), jnp.float32),
            pltpu.SemaphoreType.DMA,
            pltpu.SemaphoreType.DMA,
            pltpu.SemaphoreType.DMA,
            pltpu.SemaphoreType.DMA,
            pltpu.SemaphoreType.DMA,
            pltpu.SemaphoreType.DMA,
        ],
    )
    def _combine(posa_hbm, posb_hbm, y_hbm, out_hbm, ia_v, ib_v,
                 a0, a1, b0, b1, sa0, sa1, sb0, sb1, sw0, sw1):
        wid = lax.axis_index("s") * 2 + lax.axis_index("c")
        pltpu.sync_copy(posa_hbm.at[wid], ia_v)
        pltpu.sync_copy(posb_hbm.at[wid], ib_v)
        base = wid * _TPT
        abufs = (a0, a1)
        bbufs = (b0, b1)
        sas = (sa0, sa1)
        sbs = (sb0, sb1)
        sws = (sw0, sw1)
        gata = [None, None]
        gatb = [None, None]
        wrt = [None, None]

        def _do_chunk(c):
            q = c % 2
            gata[q].wait()
            gatb[q].wait()
            ba, bb = abufs[q], bbufs[q]

            def row_body(t, carry):
                for d in range(_F // _LANES):
                    sl = pl.ds(d * _LANES, _LANES)
                    ba[t, sl] = ba[t, sl] + bb[t, sl]
                return carry

            lax.fori_loop(0, _CCH, row_body, 0)
            wrt[q] = pltpu.async_copy(
                ba, out_hbm.at[pl.ds(base + c * _CCH, _CCH)], sws[q])

        for c in range(_NCCH):
            p = c % 2
            if wrt[p] is not None:
                wrt[p].wait()
            gata[p] = pltpu.async_copy(y_hbm.at[ia_v.at[c]], abufs[p], sas[p])
            gatb[p] = pltpu.async_copy(y_hbm.at[ib_v.at[c]], bbufs[p], sbs[p])
            if c >= 1:
                _do_chunk(c - 1)
        _do_chunk(_NCCH - 1)
        wrt[0].wait()
        wrt[1].wait()

    return _combine


def kernel(hidden_states, top_k_index, top_k_weights, w1, v1, w2):
    bsz = hidden_states.shape[0]
    hs = hidden_states.reshape(_S, _F)
    order, w_sorted, a_enc, pos_orig, meta = _route(top_k_index, top_k_weights)

    x, rw = _gather_rows_kernel()(a_enc, order, w_sorted, hs)

    w1r = w1.reshape(_E, _F, _H)
    v1r = v1.reshape(_E, _F, _H)
    w2r = w2.reshape(_E, _F, _H)
    y = _expert_matmul(meta, x, w1r, v1r, w2r, rw.reshape(_NPAD, 1))

    pos2 = pos_orig.reshape(_S, _TOPK)
    posa = pos2[:, 0].reshape(_NTILES, _NCCH, _CCH)
    posb = pos2[:, 1].reshape(_NTILES, _NCCH, _CCH)
    out = _combine_kernel()(posa, posb, y)
    return out.reshape(bsz, _S, _F)


# speedup vs baseline: 1.0949x; 1.0949x over previous
import functools

import jax
import jax.numpy as jnp
from jax import lax
from jax.experimental import pallas as pl
from jax.experimental.pallas import tpu as pltpu
from jax.experimental.pallas import tpu_sc as plsc

_E = 8
_H = 1024
_F = 1024
_TOPK = 2
_S = 2048
_A = _S * _TOPK

_BLK = 256
_NPAD = 6144
_NBLK = _NPAD // _BLK

_NTILES = 32
_RPT = _NPAD // _NTILES
_GCH = 16
_NGCH = _RPT // _GCH
_GNB = 4
_TPT = _S // _NTILES
_CCH = 16
_NCCH = _TPT // _CCH
_LANES = 16


def _route(top_k_index, top_k_weights):
    flat_e = top_k_index.reshape(-1).astype(jnp.int32)
    order = jnp.argsort(flat_e).astype(jnp.int32)
    rank = jnp.argsort(order).astype(jnp.int32)
    w_sorted = top_k_weights.reshape(-1).astype(jnp.float32)[order]

    eids = jnp.arange(_E, dtype=jnp.int32)
    cnt = jnp.sum(flat_e[None, :] == eids[:, None], axis=1).astype(jnp.int32)
    start = jnp.concatenate([jnp.zeros(1, jnp.int32), jnp.cumsum(cnt)[:-1]])
    bcnt = (cnt + _BLK - 1) // _BLK
    bends = jnp.cumsum(bcnt)
    bstart = jnp.concatenate([jnp.zeros(1, jnp.int32), bends[:-1]])
    pstart = bstart * _BLK
    pends = bends * _BLK

    pos_orig = jnp.take(pstart - start, flat_e) + rank

    used = bends[-1]
    bids = jnp.arange(_NBLK, dtype=jnp.int32)
    be_raw = jnp.minimum(jnp.sum(bids[None, :] >= bends[:, None], axis=0),
                         _E - 1).astype(jnp.int32)

    r2d = jnp.arange(_NPAD, dtype=jnp.int32).reshape(_NBLK, _BLK)
    a2d = r2d - jnp.take(pstart - start, be_raw)[:, None]
    valid2d = a2d < jnp.take(start + cnt, be_raw)[:, None]
    a2d = jnp.clip(a2d, 0, _A - 1)
    a_enc = jnp.where(valid2d, a2d,
                      -jnp.bitwise_and(r2d, _S - 1) - 1).reshape(_NPAD)
    be_last = jnp.take(be_raw, used - 1)
    be = jnp.where(bids < used, be_raw, be_last)
    xi = jnp.where(bids < used, bids, used - 1).astype(jnp.int32)
    vld = (bids < used).astype(jnp.int32)
    meta = jnp.stack([be, xi, vld])
    return order, w_sorted, a_enc, pos_orig, meta


@functools.cache
def _gather_rows_kernel():
    mesh = plsc.VectorSubcoreMesh(core_axis_name="c", subcore_axis_name="s")

    @functools.partial(
        pl.kernel,
        mesh=mesh,
        out_type=(
            jax.ShapeDtypeStruct((_NPAD, _F), jnp.float32),
            jax.ShapeDtypeStruct((_NPAD,), jnp.float32),
        ),
        scratch_types=(
            [
                pltpu.VMEM((_RPT,), jnp.int32),
                pltpu.VMEM((_RPT,), jnp.int32),
                pltpu.VMEM((_RPT,), jnp.int32),
                pltpu.VMEM((_RPT,), jnp.float32),
                pltpu.VMEM((_NGCH, _GCH), jnp.int32),
                pltpu.VMEM((_RPT,), jnp.float32),
            ]
            + [pltpu.VMEM((_GCH, _F), jnp.float32) for _ in range(_GNB)]
            + [pltpu.SemaphoreType.DMA for _ in range(2 + 2 * _GNB)]
        ),
    )
    def _gather_rows(aenc_hbm, order_hbm, ws_hbm, hs_hbm, x_hbm, rw_hbm,
                     aenc_v, apos_v, o_v, wv_v, idx_v, rw_v, *bufsem):
        bufs = bufsem[:_GNB]
        se0, se1 = bufsem[_GNB], bufsem[_GNB + 1]
        sgs = bufsem[_GNB + 2:2 * _GNB + 2]
        sws = bufsem[2 * _GNB + 2:]
        wid = lax.axis_index("s") * 2 + lax.axis_index("c")
        base = wid * _RPT

        pltpu.sync_copy(aenc_hbm.at[pl.ds(base, _RPT)], aenc_v)
        for c in range(_NGCH):
            sl = pl.ds(c * _GCH, _GCH)
            apos_v[sl] = jnp.maximum(aenc_v[sl], 0)
        cp0 = pltpu.async_copy(order_hbm.at[apos_v], o_v, se0)
        cp1 = pltpu.async_copy(ws_hbm.at[apos_v], wv_v, se1)
        cp0.wait()
        cp1.wait()
        for c in range(_NGCH):
            sl = pl.ds(c * _GCH, _GCH)
            av = aenc_v[sl]
            vld = av >= 0
            idx_v[c, :] = jnp.where(vld, jnp.right_shift(o_v[sl], 1), -av - 1)
            rw_v[sl] = jnp.where(vld, wv_v[sl], 0.0)
        pltpu.sync_copy(rw_v, rw_hbm.at[pl.ds(base, _RPT)])

        gat = [None] * _GNB
        wrt = [None] * _GNB

        def _write(c):
            q = c % _GNB
            gat[q].wait()
            wrt[q] = pltpu.async_copy(
                bufs[q], x_hbm.at[pl.ds(base + c * _GCH, _GCH)], sws[q])

        for c in range(_NGCH):
            p = c % _GNB
            if wrt[p] is not None:
                wrt[p].wait()
            gat[p] = pltpu.async_copy(hs_hbm.at[idx_v.at[c]], bufs[p], sgs[p])
            if c >= 2:
                _write(c - 2)
        _write(_NGCH - 2)
        _write(_NGCH - 1)
        for p in range(_GNB):
            if wrt[p] is not None:
                wrt[p].wait()

    return _gather_rows


def _expert_body(meta_ref, x_ref, w1_ref, v1_ref, w2_ref, rw_ref, y_ref):
    i = pl.program_id(0)

    @pl.when(meta_ref[2, i] == 1)
    def _():
        x = x_ref[...]
        g = jnp.dot(x, w1_ref[0], preferred_element_type=jnp.float32)
        u = jnp.dot(x, v1_ref[0], preferred_element_type=jnp.float32)
        inter = (g * jax.nn.sigmoid(g)) * u
        y = lax.dot_general(inter, w2_ref[0], (((1,), (1,)), ((), ())),
                            preferred_element_type=jnp.float32)
        y_ref[...] = y * rw_ref[...]


def _expert_matmul(meta, x, w1r, v1r, w2r, rw):
    grid_spec = pltpu.PrefetchScalarGridSpec(
        num_scalar_prefetch=1,
        grid=(_NBLK,),
        in_specs=[
            pl.BlockSpec((_BLK, _F), lambda i, m: (m[1, i], 0)),
            pl.BlockSpec((1, _F, _H), lambda i, m: (m[0, i], 0, 0)),
            pl.BlockSpec((1, _F, _H), lambda i, m: (m[0, i], 0, 0)),
            pl.BlockSpec((1, _F, _H), lambda i, m: (m[0, i], 0, 0)),
            pl.BlockSpec((_BLK, 1), lambda i, m: (m[1, i], 0)),
        ],
        out_specs=pl.BlockSpec((_BLK, _F), lambda i, m: (i, 0)),
    )
    return pl.pallas_call(
        _expert_body,
        grid_spec=grid_spec,
        out_shape=jax.ShapeDtypeStruct((_NPAD, _F), jnp.float32),
    )(meta, x, w1r, v1r, w2r, rw)


@functools.cache
def _combine_kernel():
    mesh = plsc.VectorSubcoreMesh(core_axis_name="c", subcore_axis_name="s")

    @functools.partial(
        pl.kernel,
        mesh=mesh,
        out_type=jax.ShapeDtypeStruct((_S, _F), jnp.float32),
        scratch_types=[
            pltpu.VMEM((_NCCH, _CCH), jnp.int32),
            pltpu.VMEM((_NCCH, _CCH), jnp.int32),
            pltpu.VMEM((_CCH, _F), jnp.float32),
            pltpu.VMEM((_CCH, _F), jnp.float32),
            pltpu.VMEM((_CCH, _F), jnp.float32),
            pltpu.VMEM((_CCH, _F), jnp.float32),
            pltpu.SemaphoreType.DMA,
            pltpu.SemaphoreType.DMA,
            pltpu.SemaphoreType.DMA,
            pltpu.SemaphoreType.DMA,
            pltpu.SemaphoreType.DMA,
            pltpu.SemaphoreType.DMA,
        ],
    )
    def _combine(posa_hbm, posb_hbm, y_hbm, out_hbm, ia_v, ib_v,
                 a0, a1, b0, b1, sa0, sa1, sb0, sb1, sw0, sw1):
        wid = lax.axis_index("s") * 2 + lax.axis_index("c")
        pltpu.sync_copy(posa_hbm.at[wid], ia_v)
        pltpu.sync_copy(posb_hbm.at[wid], ib_v)
        base = wid * _TPT
        abufs = (a0, a1)
        bbufs = (b0, b1)
        sas = (sa0, sa1)
        sbs = (sb0, sb1)
        sws = (sw0, sw1)
        gata = [None, None]
        gatb = [None, None]
        wrt = [None, None]

        def _do_chunk(c):
            q = c % 2
            gata[q].wait()
            gatb[q].wait()
            ba, bb = abufs[q], bbufs[q]

            def row_body(t, carry):
                for d in range(_F // _LANES):
                    sl = pl.ds(d * _LANES, _LANES)
                    ba[t, sl] = ba[t, sl] + bb[t, sl]
                return carry

            lax.fori_loop(0, _CCH, row_body, 0)
            wrt[q] = pltpu.async_copy(
                ba, out_hbm.at[pl.ds(base + c * _CCH, _CCH)], sws[q])

        for c in range(_NCCH):
            p = c % 2
            if wrt[p] is not None:
                wrt[p].wait()
            gata[p] = pltpu.async_copy(y_hbm.at[ia_v.at[c]], abufs[p], sas[p])
            gatb[p] = pltpu.async_copy(y_hbm.at[ib_v.at[c]], bbufs[p], sbs[p])
            if c >= 1:
                _do_chunk(c - 1)
        _do_chunk(_NCCH - 1)
        wrt[0].wait()
        wrt[1].wait()

    return _combine


def kernel(hidden_states, top_k_index, top_k_weights, w1, v1, w2):
    bsz = hidden_states.shape[0]
    hs = hidden_states.reshape(_S, _F)
    order, w_sorted, a_enc, pos_orig, meta = _route(top_k_index, top_k_weights)

    x, rw = _gather_rows_kernel()(a_enc, order, w_sorted, hs)

    w1r = w1.reshape(_E, _F, _H)
    v1r = v1.reshape(_E, _F, _H)
    w2r = w2.reshape(_E, _F, _H)
    y = _expert_matmul(meta, x, w1r, v1r, w2r, rw.reshape(_NPAD, 1))

    pos2 = pos_orig.reshape(_S, _TOPK)
    posa = pos2[:, 0].reshape(_NTILES, _NCCH, _CCH)
    posb = pos2[:, 1].reshape(_NTILES, _NCCH, _CCH)
    out = _combine_kernel()(posa, posb, y)
    return out.reshape(bsz, _S, _F)


# speedup vs baseline: 1.0958x; 1.0008x over previous
import functools

import jax
import jax.numpy as jnp
from jax import lax
from jax.experimental import pallas as pl
from jax.experimental.pallas import tpu as pltpu
from jax.experimental.pallas import tpu_sc as plsc

_E = 8
_H = 1024
_F = 1024
_TOPK = 2
_S = 2048
_A = _S * _TOPK

_BLK = 256
_NPAD = 6144
_NBLK = _NPAD // _BLK

_NTILES = 32
_RPT = _NPAD // _NTILES
_GCH = 16
_NGCH = _RPT // _GCH
_GNB = 4
_TPT = _S // _NTILES
_CCH = 16
_NCCH = _TPT // _CCH
_LANES = 16


def _route(top_k_index, top_k_weights):
    flat_e = top_k_index.reshape(-1).astype(jnp.int32)
    order = jnp.argsort(flat_e).astype(jnp.int32)
    rank = jnp.argsort(order).astype(jnp.int32)
    w_sorted = top_k_weights.reshape(-1).astype(jnp.float32)[order]

    eids = jnp.arange(_E, dtype=jnp.int32)
    cnt = jnp.sum(flat_e[None, :] == eids[:, None], axis=1).astype(jnp.int32)
    start = jnp.concatenate([jnp.zeros(1, jnp.int32), jnp.cumsum(cnt)[:-1]])
    bcnt = (cnt + _BLK - 1) // _BLK
    bends = jnp.cumsum(bcnt)
    bstart = jnp.concatenate([jnp.zeros(1, jnp.int32), bends[:-1]])
    pstart = bstart * _BLK
    pends = bends * _BLK

    pos_orig = jnp.take(pstart - start, flat_e) + rank

    used = bends[-1]
    bids = jnp.arange(_NBLK, dtype=jnp.int32)
    be_raw = jnp.minimum(jnp.sum(bids[None, :] >= bends[:, None], axis=0),
                         _E - 1).astype(jnp.int32)

    r2d = jnp.arange(_NPAD, dtype=jnp.int32).reshape(_NBLK, _BLK)
    a2d = r2d - jnp.take(pstart - start, be_raw)[:, None]
    valid2d = a2d < jnp.take(start + cnt, be_raw)[:, None]
    a2d = jnp.clip(a2d, 0, _A - 1)
    a_enc = jnp.where(valid2d, a2d,
                      -jnp.bitwise_and(r2d, _S - 1) - 1).reshape(_NPAD)
    be_last = jnp.take(be_raw, used - 1)
    be = jnp.where(bids < used, be_raw, be_last)
    xi = jnp.where(bids < used, bids, used - 1).astype(jnp.int32)
    vld = (bids < used).astype(jnp.int32)
    meta = jnp.stack([be, xi, vld])
    return order, w_sorted, a_enc, pos_orig, meta


@functools.cache
def _gather_rows_kernel():
    mesh = plsc.VectorSubcoreMesh(core_axis_name="c", subcore_axis_name="s")

    @functools.partial(
        pl.kernel,
        mesh=mesh,
        out_type=(
            jax.ShapeDtypeStruct((_NPAD, _F), jnp.float32),
            jax.ShapeDtypeStruct((_NPAD,), jnp.float32),
        ),
        scratch_types=(
            [
                pltpu.VMEM((_RPT,), jnp.int32),
                pltpu.VMEM((_RPT,), jnp.int32),
                pltpu.VMEM((_RPT,), jnp.int32),
                pltpu.VMEM((_RPT,), jnp.float32),
                pltpu.VMEM((_NGCH, _GCH), jnp.int32),
                pltpu.VMEM((_RPT,), jnp.float32),
            ]
            + [pltpu.VMEM((_GCH, _F), jnp.float32) for _ in range(_GNB)]
            + [pltpu.SemaphoreType.DMA for _ in range(4 + 2 * _GNB)]
        ),
    )
    def _gather_rows(aenc_hbm, order_hbm, ws_hbm, hs_hbm, x_hbm, rw_hbm,
                     aenc_v, apos_v, o_v, wv_v, idx_v, rw_v, *bufsem):
        bufs = bufsem[:_GNB]
        ses = bufsem[_GNB:_GNB + 4]
        sgs = bufsem[_GNB + 4:2 * _GNB + 4]
        sws = bufsem[2 * _GNB + 4:]
        wid = lax.axis_index("s") * 2 + lax.axis_index("c")
        base = wid * _RPT

        pltpu.sync_copy(aenc_hbm.at[pl.ds(base, _RPT)], aenc_v)
        for c in range(_NGCH):
            sl = pl.ds(c * _GCH, _GCH)
            apos_v[sl] = jnp.maximum(aenc_v[sl], 0)
        half = _RPT // 2
        lo, hi = pl.ds(0, half), pl.ds(half, half)
        cps = [
            pltpu.async_copy(order_hbm.at[apos_v.at[lo]], o_v.at[lo], ses[0]),
            pltpu.async_copy(order_hbm.at[apos_v.at[hi]], o_v.at[hi], ses[1]),
            pltpu.async_copy(ws_hbm.at[apos_v.at[lo]], wv_v.at[lo], ses[2]),
            pltpu.async_copy(ws_hbm.at[apos_v.at[hi]], wv_v.at[hi], ses[3]),
        ]
        for cp in cps:
            cp.wait()
        for c in range(_NGCH):
            sl = pl.ds(c * _GCH, _GCH)
            av = aenc_v[sl]
            vld = av >= 0
            idx_v[c, :] = jnp.where(vld, jnp.right_shift(o_v[sl], 1), -av - 1)
            rw_v[sl] = jnp.where(vld, wv_v[sl], 0.0)
        pltpu.sync_copy(rw_v, rw_hbm.at[pl.ds(base, _RPT)])

        gat = [None] * _GNB
        wrt = [None] * _GNB

        def _write(c):
            q = c % _GNB
            gat[q].wait()
            wrt[q] = pltpu.async_copy(
                bufs[q], x_hbm.at[pl.ds(base + c * _GCH, _GCH)], sws[q])

        for c in range(_NGCH):
            p = c % _GNB
            if wrt[p] is not None:
                wrt[p].wait()
            gat[p] = pltpu.async_copy(hs_hbm.at[idx_v.at[c]], bufs[p], sgs[p])
            if c >= 2:
                _write(c - 2)
        _write(_NGCH - 2)
        _write(_NGCH - 1)
        for p in range(_GNB):
            if wrt[p] is not None:
                wrt[p].wait()

    return _gather_rows


def _expert_body(meta_ref, x_ref, w1_ref, v1_ref, w2_ref, rw_ref, y_ref):
    i = pl.program_id(0)

    @pl.when(meta_ref[2, i] == 1)
    def _():
        x = x_ref[...]
        g = jnp.dot(x, w1_ref[0], preferred_element_type=jnp.float32)
        u = jnp.dot(x, v1_ref[0], preferred_element_type=jnp.float32)
        inter = (g * jax.nn.sigmoid(g)) * u
        y = lax.dot_general(inter, w2_ref[0], (((1,), (1,)), ((), ())),
                            preferred_element_type=jnp.float32)
        y_ref[...] = y * rw_ref[...]


def _expert_matmul(meta, x, w1r, v1r, w2r, rw):
    grid_spec = pltpu.PrefetchScalarGridSpec(
        num_scalar_prefetch=1,
        grid=(_NBLK,),
        in_specs=[
            pl.BlockSpec((_BLK, _F), lambda i, m: (m[1, i], 0)),
            pl.BlockSpec((1, _F, _H), lambda i, m: (m[0, i], 0, 0)),
            pl.BlockSpec((1, _F, _H), lambda i, m: (m[0, i], 0, 0)),
            pl.BlockSpec((1, _F, _H), lambda i, m: (m[0, i], 0, 0)),
            pl.BlockSpec((_BLK, 1), lambda i, m: (m[1, i], 0)),
        ],
        out_specs=pl.BlockSpec((_BLK, _F), lambda i, m: (i, 0)),
    )
    return pl.pallas_call(
        _expert_body,
        grid_spec=grid_spec,
        out_shape=jax.ShapeDtypeStruct((_NPAD, _F), jnp.float32),
    )(meta, x, w1r, v1r, w2r, rw)


@functools.cache
def _combine_kernel():
    mesh = plsc.VectorSubcoreMesh(core_axis_name="c", subcore_axis_name="s")

    @functools.partial(
        pl.kernel,
        mesh=mesh,
        out_type=jax.ShapeDtypeStruct((_S, _F), jnp.float32),
        scratch_types=[
            pltpu.VMEM((_NCCH, _CCH), jnp.int32),
            pltpu.VMEM((_NCCH, _CCH), jnp.int32),
            pltpu.VMEM((_CCH, _F), jnp.float32),
            pltpu.VMEM((_CCH, _F), jnp.float32),
            pltpu.VMEM((_CCH, _F), jnp.float32),
            pltpu.VMEM((_CCH, _F), jnp.float32),
            pltpu.SemaphoreType.DMA,
            pltpu.SemaphoreType.DMA,
            pltpu.SemaphoreType.DMA,
            pltpu.SemaphoreType.DMA,
            pltpu.SemaphoreType.DMA,
            pltpu.SemaphoreType.DMA,
        ],
    )
    def _combine(posa_hbm, posb_hbm, y_hbm, out_hbm, ia_v, ib_v,
                 a0, a1, b0, b1, sa0, sa1, sb0, sb1, sw0, sw1):
        wid = lax.axis_index("s") * 2 + lax.axis_index("c")
        pltpu.sync_copy(posa_hbm.at[wid], ia_v)
        pltpu.sync_copy(posb_hbm.at[wid], ib_v)
        base = wid * _TPT
        abufs = (a0, a1)
        bbufs = (b0, b1)
        sas = (sa0, sa1)
        sbs = (sb0, sb1)
        sws = (sw0, sw1)
        gata = [None, None]
        gatb = [None, None]
        wrt = [None, None]

        def _do_chunk(c):
            q = c % 2
            gata[q].wait()
            gatb[q].wait()
            ba, bb = abufs[q], bbufs[q]

            def row_body(t, carry):
                for d in range(_F // _LANES):
                    sl = pl.ds(d * _LANES, _LANES)
                    ba[t, sl] = ba[t, sl] + bb[t, sl]
                return carry

            lax.fori_loop(0, _CCH, row_body, 0)
            wrt[q] = pltpu.async_copy(
                ba, out_hbm.at[pl.ds(base + c * _CCH, _CCH)], sws[q])

        for c in range(_NCCH):
            p = c % 2
            if wrt[p] is not None:
                wrt[p].wait()
            gata[p] = pltpu.async_copy(y_hbm.at[ia_v.at[c]], abufs[p], sas[p])
            gatb[p] = pltpu.async_copy(y_hbm.at[ib_v.at[c]], bbufs[p], sbs[p])
            if c >= 1:
                _do_chunk(c - 1)
        _do_chunk(_NCCH - 1)
        wrt[0].wait()
        wrt[1].wait()

    return _combine


def kernel(hidden_states, top_k_index, top_k_weights, w1, v1, w2):
    bsz = hidden_states.shape[0]
    hs = hidden_states.reshape(_S, _F)
    order, w_sorted, a_enc, pos_orig, meta = _route(top_k_index, top_k_weights)

    x, rw = _gather_rows_kernel()(a_enc, order, w_sorted, hs)

    w1r = w1.reshape(_E, _F, _H)
    v1r = v1.reshape(_E, _F, _H)
    w2r = w2.reshape(_E, _F, _H)
    y = _expert_matmul(meta, x, w1r, v1r, w2r, rw.reshape(_NPAD, 1))

    pos2 = pos_orig.reshape(_S, _TOPK)
    posa = pos2[:, 0].reshape(_NTILES, _NCCH, _CCH)
    posb = pos2[:, 1].reshape(_NTILES, _NCCH, _CCH)
    out = _combine_kernel()(posa, posb, y)
    return out.reshape(bsz, _S, _F)
